# baseline (speedup 1.0000x reference)
import functools

import jax
import jax.numpy as jnp
from jax import lax
from jax.experimental import pallas as pl
from jax.experimental.pallas import tpu as pltpu
from jax.experimental.pallas import tpu_sc as plsc

_TBG = 256
_RB = 512
_NC, _NS = 2, 16
_CH_D = 32
_CH_G = 32


def _dot_t(a, b):
    return lax.dot_general(a, b, (((1,), (1,)), ((), ())),
                           preferred_element_type=jnp.float32)


def _meta_body(n_exp, tbg, nb, x_ref, wg_ref, dest_ref, m1_ref,
               m2_ref, be_ref, used_ref):
    x = x_ref[...]
    t = x.shape[0]
    s = _dot_t(x, wg_ref[...])
    s = s - jnp.max(s, axis=1, keepdims=True)
    s = jnp.exp(s)
    s = s / jnp.sum(s, axis=1, keepdims=True)
    iota = lax.broadcasted_iota(jnp.int32, s.shape, 1)
    m1 = jnp.max(s, axis=1, keepdims=True)
    i1 = jnp.min(jnp.where(s == m1, iota, n_exp), axis=1, keepdims=True)
    s2 = jnp.where(iota == i1, -jnp.inf, s)
    m2 = jnp.max(s2, axis=1, keepdims=True)
    i2 = jnp.min(jnp.where(s2 == m2, iota, n_exp), axis=1, keepdims=True)
    m1_ref[...] = m1
    m2_ref[...] = m2

    ep = jnp.concatenate([i1, i2], axis=0)
    rb = _RB
    nblk = (2 * t) // rb
    ri = lax.broadcasted_iota(jnp.int32, (rb, rb), 0)
    ci = lax.broadcasted_iota(jnp.int32, (rb, rb), 1)
    ltri = (ri > ci).astype(jnp.float32)
    lane = lax.broadcasted_iota(jnp.int32, (rb, n_exp), 1)
    ohs, prs, sums = [], [], []
    for b in range(nblk):
        ohb = (lane == ep[b * rb:(b + 1) * rb, :]).astype(jnp.float32)
        ohs.append(ohb)
        prs.append(lax.dot_general(ltri, ohb, (((1,), (0,)), ((), ())),
                                   preferred_element_type=jnp.float32))
        sums.append(jnp.sum(ohb, axis=0, keepdims=True))
    bs = jnp.concatenate(sums, axis=0)
    bri = lax.broadcasted_iota(jnp.int32, (nblk, nblk), 0)
    bci = lax.broadcasted_iota(jnp.int32, (nblk, nblk), 1)
    btri = (bri > bci).astype(jnp.float32)
    boff = lax.dot_general(btri, bs, (((1,), (0,)), ((), ())),
                           preferred_element_type=jnp.float32)
    counts = jnp.sum(bs, axis=0, keepdims=True)
    nbe = (counts.astype(jnp.int32) + (tbg - 1)) // tbg
    eri = lax.broadcasted_iota(jnp.int32, (n_exp, n_exp), 0)
    eci = lax.broadcasted_iota(jnp.int32, (n_exp, n_exp), 1)
    ustri = (eri < eci).astype(jnp.float32)
    startb = lax.dot_general(nbe.astype(jnp.float32), ustri,
                             (((1,), (0,)), ((), ())),
                             preferred_element_type=jnp.float32)
    pbase = startb * tbg
    used_ref[...] = jnp.sum(nbe, axis=1, keepdims=True)
    for b in range(nblk):
        pos = prs[b] + boff[b:b + 1, :] + pbase
        destb = jnp.sum(ohs[b] * pos, axis=1, keepdims=True)
        dest_ref[b * rb:(b + 1) * rb, :] = destb.astype(jnp.int32)
    bi = lax.broadcasted_iota(jnp.int32, (nb, n_exp), 0).astype(jnp.float32)
    be_ref[...] = (jnp.sum((bi >= startb).astype(jnp.int32), axis=1,
                           keepdims=True) - 1)


def _meta(x, wg, tbg, nb, interpret=False):
    t, d = x.shape
    n_exp = wg.shape[0]
    return pl.pallas_call(
        functools.partial(_meta_body, n_exp, tbg, nb),
        out_shape=(
            jax.ShapeDtypeStruct((2 * t, 1), jnp.int32),
            jax.ShapeDtypeStruct((t, 1), jnp.float32),
            jax.ShapeDtypeStruct((t, 1), jnp.float32),
            jax.ShapeDtypeStruct((nb, 1), jnp.int32),
            jax.ShapeDtypeStruct((1, 1), jnp.int32),
        ),
        interpret=interpret,
    )(x, wg)


def _sc_dispatch(x3, dest, tok, pt):
    p2 = dest.shape[0]
    nw = _NC * _NS
    rows_pw = p2 // nw
    nch = rows_pw // _CH_D
    mesh = plsc.VectorSubcoreMesh(core_axis_name="c", subcore_axis_name="s")

    @functools.partial(
        pl.kernel, mesh=mesh,
        out_type=jax.ShapeDtypeStruct((pt,) + x3.shape[1:], x3.dtype),
        scratch_types=[
            pltpu.VMEM((rows_pw,), jnp.int32),
            pltpu.VMEM((_CH_D,), jnp.int32),
            pltpu.VMEM((_CH_D,), jnp.int32),
            pltpu.VMEM((_CH_D,) + x3.shape[1:], x3.dtype),
            pltpu.VMEM((_CH_D,) + x3.shape[1:], x3.dtype),
            pltpu.SemaphoreType.DMA,
            pltpu.SemaphoreType.DMA,
            pltpu.SemaphoreType.DMA,
            pltpu.SemaphoreType.DMA,
        ])
    def k(x_hbm, dest_hbm, tok_hbm, xs_hbm, tchunk, dc0, dc1, rb0, rb1,
          sg0, sg1, ss0, ss1):
        cid = lax.axis_index("c")
        sid = lax.axis_index("s")
        wid = sid * _NC + cid
        rowbase = wid * rows_pw
        pltpu.sync_copy(tok_hbm.at[pl.ds(rowbase, rows_pw)], tchunk)
        bufs = [(dc0, rb0, sg0, ss0), (dc1, rb1, sg1, ss1)]

        def issue_gather(kk):
            dc, rb, sg, _ = bufs[kk % 2]
            pltpu.sync_copy(dest_hbm.at[pl.ds(rowbase + kk * _CH_D, _CH_D)], dc)
            return pltpu.async_copy(
                x_hbm.at[tchunk.at[pl.ds(kk * _CH_D, _CH_D)]], rb, sg)

        gathers = {0: issue_gather(0)}
        if nch > 1:
            gathers[1] = issue_gather(1)
        scatters = {}
        for kk in range(nch):
            dc, rb, _, ss = bufs[kk % 2]
            gathers[kk].wait()
            scatters[kk] = pltpu.async_copy(rb, xs_hbm.at[dc], ss)
            if kk + 2 < nch:
                scatters[kk].wait()
                gathers[kk + 2] = issue_gather(kk + 2)
        for kk in range(max(0, nch - 2), nch):
            scatters[kk].wait()

    return k(x3, dest, tok)


def _sc_gather(ys3, dest):
    p2 = dest.shape[0]
    nw = _NC * _NS
    rows_pw = p2 // nw
    nch = rows_pw // _CH_G
    mesh = plsc.VectorSubcoreMesh(core_axis_name="c", subcore_axis_name="s")

    @functools.partial(
        pl.kernel, mesh=mesh,
        out_type=jax.ShapeDtypeStruct((p2,) + ys3.shape[1:], ys3.dtype),
        scratch_types=[
            pltpu.VMEM((rows_pw,), jnp.int32),
            pltpu.VMEM((_CH_G,) + ys3.shape[1:], ys3.dtype),
            pltpu.VMEM((_CH_G,) + ys3.shape[1:], ys3.dtype),
            pltpu.SemaphoreType.DMA,
            pltpu.SemaphoreType.DMA,
            pltpu.SemaphoreType.DMA,
            pltpu.SemaphoreType.DMA,
        ])
    def k(ys_hbm, dest_hbm, yg_hbm, idx_v, rb0, rb1, sg0, sg1, ss0, ss1):
        cid = lax.axis_index("c")
        sid = lax.axis_index("s")
        wid = sid * _NC + cid
        rowbase = wid * rows_pw
        pltpu.sync_copy(dest_hbm.at[pl.ds(rowbase, rows_pw)], idx_v)
        bufs = [(rb0, sg0, ss0), (rb1, sg1, ss1)]

        def issue_gather(kk):
            rb, sg, _ = bufs[kk % 2]
            return pltpu.async_copy(
                ys_hbm.at[idx_v.at[pl.ds(kk * _CH_G, _CH_G)]], rb, sg)

        gathers = {0: issue_gather(0)}
        if nch > 1:
            gathers[1] = issue_gather(1)
        writes = {}
        for kk in range(nch):
            rb, _, ss = bufs[kk % 2]
            gathers[kk].wait()
            writes[kk] = pltpu.async_copy(
                rb, yg_hbm.at[pl.ds(rowbase + kk * _CH_G, _CH_G)], ss)
            if kk + 2 < nch:
                writes[kk].wait()
                gathers[kk + 2] = issue_gather(kk + 2)
        for kk in range(max(0, nch - 2), nch):
            writes[kk].wait()

    return k(ys3, dest)


def _wcast_body(w1_ref, w3_ref, w2_ref, ws1_ref, ws3_ref, ws2_ref,
                o1_ref, o3_ref, o2_ref, os1_ref, os3_ref, os2_ref):
    e = pl.program_id(0)
    o1_ref[...] = w1_ref[...].astype(jnp.bfloat16)
    o3_ref[...] = w3_ref[...].astype(jnp.bfloat16)
    o2_ref[...] = w2_ref[...].astype(jnp.bfloat16)

    @pl.when(e == 0)
    def _():
        os1_ref[...] = ws1_ref[...].astype(jnp.bfloat16)
        os3_ref[...] = ws3_ref[...].astype(jnp.bfloat16)
        os2_ref[...] = ws2_ref[...].astype(jnp.bfloat16)


def _wcast(w1, w3, w2, ws1, ws3, ws2, interpret=False):
    n_exp, h, d = w1.shape
    sh_h = ws1.shape[0]
    ee = lambda e: (e, 0, 0)
    c2 = lambda e: (0, 0)
    return pl.pallas_call(
        _wcast_body,
        grid=(n_exp,),
        in_specs=[
            pl.BlockSpec((1, h, d), ee),
            pl.BlockSpec((1, h, d), ee),
            pl.BlockSpec((1, d, h), ee),
            pl.BlockSpec((sh_h, d), c2),
            pl.BlockSpec((sh_h, d), c2),
            pl.BlockSpec((d, sh_h), c2),
        ],
        out_specs=[
            pl.BlockSpec((1, h, d), ee),
            pl.BlockSpec((1, h, d), ee),
            pl.BlockSpec((1, d, h), ee),
            pl.BlockSpec((sh_h, d), c2),
            pl.BlockSpec((sh_h, d), c2),
            pl.BlockSpec((d, sh_h), c2),
        ],
        out_shape=(
            jax.ShapeDtypeStruct((n_exp, h, d), jnp.bfloat16),
            jax.ShapeDtypeStruct((n_exp, h, d), jnp.bfloat16),
            jax.ShapeDtypeStruct((n_exp, d, h), jnp.bfloat16),
            jax.ShapeDtypeStruct((sh_h, d), jnp.bfloat16),
            jax.ShapeDtypeStruct((sh_h, d), jnp.bfloat16),
            jax.ShapeDtypeStruct((d, sh_h), jnp.bfloat16),
        ),
        interpret=interpret,
    )(w1, w3, w2, ws1, ws3, ws2)


def _gemm_body(tbg, ncc, sp_ref, xs_ref, w1_ref, w3_ref, w2_ref, ys_ref):
    b = pl.program_id(0)
    d = ncc * 128

    @pl.when(b < sp_ref[0])
    def _():
        xt = jnp.reshape(xs_ref[...], (tbg, d)).astype(jnp.bfloat16)
        h1 = _dot_t(xt, w1_ref[0])
        h3 = _dot_t(xt, w3_ref[0])
        g = (h1 * lax.logistic(h1) * h3).astype(jnp.bfloat16)
        ys_ref[...] = jnp.reshape(_dot_t(g, w2_ref[0]), (tbg, ncc, 128))


def _gemm(sp, xs3, w1b, w3b, w2b, tbg, interpret=False):
    pt = xs3.shape[0]
    n_exp, h, d = w1b.shape
    ncc = d // 128
    nb = pt // tbg
    grid_spec = pltpu.PrefetchScalarGridSpec(
        num_scalar_prefetch=1,
        grid=(nb,),
        in_specs=[
            pl.BlockSpec((tbg, ncc, 128), lambda b, sp: (b, 0, 0)),
            pl.BlockSpec((1, h, d), lambda b, sp: (sp[b + 1], 0, 0)),
            pl.BlockSpec((1, h, d), lambda b, sp: (sp[b + 1], 0, 0)),
            pl.BlockSpec((1, d, h), lambda b, sp: (sp[b + 1], 0, 0)),
        ],
        out_specs=pl.BlockSpec((tbg, ncc, 128), lambda b, sp: (b, 0, 0)),
    )
    return pl.pallas_call(
        functools.partial(_gemm_body, tbg, ncc),
        grid_spec=grid_spec,
        out_shape=jax.ShapeDtypeStruct((pt, ncc, 128), jnp.float32),
        interpret=interpret,
    )(sp, xs3, w1b, w3b, w2b)


def _shared_body(x_ref, ws1_ref, ws3_ref, ws2_ref, sh_ref):
    xt = x_ref[...].astype(jnp.bfloat16)
    h1 = _dot_t(xt, ws1_ref[...])
    h3 = _dot_t(xt, ws3_ref[...])
    g = (h1 * lax.logistic(h1) * h3).astype(jnp.bfloat16)
    sh_ref[...] = _dot_t(g, ws2_ref[...]).astype(jnp.bfloat16)


def _shared(xb, ws1b, ws3b, ws2b, interpret=False):
    t, d = xb.shape
    sh_h = ws1b.shape[0]
    tb = min(1024, t)
    return pl.pallas_call(
        _shared_body,
        grid=(t // tb,),
        in_specs=[
            pl.BlockSpec((tb, d), lambda i: (i, 0)),
            pl.BlockSpec((sh_h, d), lambda i: (0, 0)),
            pl.BlockSpec((sh_h, d), lambda i: (0, 0)),
            pl.BlockSpec((d, sh_h), lambda i: (0, 0)),
        ],
        out_specs=pl.BlockSpec((tb, d), lambda i: (i, 0)),
        out_shape=jax.ShapeDtypeStruct((t, d), jnp.bfloat16),
        interpret=interpret,
    )(xb, ws1b, ws3b, ws2b)


def _combine_body(tb, d, ya_ref, yb_ref, m1_ref, m2_ref, x_ref, ws1_ref,
                  ws3_ref, ws2_ref, z_ref):
    xt = x_ref[...].astype(jnp.bfloat16)
    h1 = _dot_t(xt, ws1_ref[...])
    h3 = _dot_t(xt, ws3_ref[...])
    g = (h1 * lax.logistic(h1) * h3).astype(jnp.bfloat16)
    sh = _dot_t(g, ws2_ref[...])
    ya = jnp.reshape(ya_ref[...], (tb, d))
    yb = jnp.reshape(yb_ref[...], (tb, d))
    z_ref[...] = m1_ref[...] * ya + m2_ref[...] * yb + sh


def _combine(yg3, m1, m2, x, ws1b, ws3b, ws2b, interpret=False):
    t, d = x.shape
    sh_h = ws1b.shape[0]
    ncc = d // 128
    tb = min(1024, t)
    nts = t // tb
    return pl.pallas_call(
        functools.partial(_combine_body, tb, d),
        grid=(nts,),
        in_specs=[
            pl.BlockSpec((tb, ncc, 128), lambda i: (i, 0, 0)),
            pl.BlockSpec((tb, ncc, 128), lambda i, _n=nts: (i + _n, 0, 0)),
            pl.BlockSpec((tb, 1), lambda i: (i, 0)),
            pl.BlockSpec((tb, 1), lambda i: (i, 0)),
            pl.BlockSpec((tb, d), lambda i: (i, 0)),
            pl.BlockSpec((sh_h, d), lambda i: (0, 0)),
            pl.BlockSpec((sh_h, d), lambda i: (0, 0)),
            pl.BlockSpec((d, sh_h), lambda i: (0, 0)),
        ],
        out_specs=pl.BlockSpec((tb, d), lambda i: (i, 0)),
        out_shape=jax.ShapeDtypeStruct((t, d), jnp.float32),
        interpret=interpret,
    )(yg3, yg3, m1, m2, x, ws1b, ws3b, ws2b)


def kernel(x, Wg, w1, w3, w2, ws1, ws3, ws2):
    t, d = x.shape
    n_exp = Wg.shape[0]
    k_act = 2
    p2 = k_act * t
    pt = p2 + n_exp * _TBG
    nb = pt // _TBG

    dest, m1, m2, be, used = _meta(x, Wg, _TBG, nb)
    w1b, w3b, w2b, ws1b, ws3b, ws2b = _wcast(w1, w3, w2, ws1, ws3, ws2)
    sp = jnp.concatenate([used.reshape(-1), be.reshape(-1)])
    tok = jnp.concatenate([jnp.arange(t, dtype=jnp.int32)] * k_act)
    x3 = x.reshape(t, d // 128, 128)
    xs3 = _sc_dispatch(x3, dest.reshape(-1), tok, pt)
    ys3 = _gemm(sp, xs3, w1b, w3b, w2b, _TBG)
    yg3 = _sc_gather(ys3, dest.reshape(-1))
    return _combine(yg3, m1, m2, x, ws1b, ws3b, ws2b)


# speedup vs baseline: 1.1004x; 1.1004x over previous
import functools

import jax
import jax.numpy as jnp
from jax import lax
from jax.experimental import pallas as pl
from jax.experimental.pallas import tpu as pltpu
from jax.experimental.pallas import tpu_sc as plsc

_TBG = 256
_RB = 512
_NC, _NS = 2, 16
_CH_D = 32
_CH_G = 32


def _dot_t(a, b):
    return lax.dot_general(a, b, (((1,), (1,)), ((), ())),
                           preferred_element_type=jnp.float32)


def _meta_body(n_exp, tbg, nb, x_ref, wg_ref, dest_ref, m1_ref,
               m2_ref, be_ref, used_ref):
    x = x_ref[...]
    t = x.shape[0]
    s = _dot_t(x, wg_ref[...])
    s = s - jnp.max(s, axis=1, keepdims=True)
    s = jnp.exp(s)
    s = s / jnp.sum(s, axis=1, keepdims=True)
    iota = lax.broadcasted_iota(jnp.int32, s.shape, 1)
    m1 = jnp.max(s, axis=1, keepdims=True)
    i1 = jnp.min(jnp.where(s == m1, iota, n_exp), axis=1, keepdims=True)
    s2 = jnp.where(iota == i1, -jnp.inf, s)
    m2 = jnp.max(s2, axis=1, keepdims=True)
    i2 = jnp.min(jnp.where(s2 == m2, iota, n_exp), axis=1, keepdims=True)
    m1_ref[...] = m1
    m2_ref[...] = m2

    ep = jnp.concatenate([i1, i2], axis=0)
    rb = _RB
    nblk = (2 * t) // rb
    ri = lax.broadcasted_iota(jnp.int32, (rb, rb), 0)
    ci = lax.broadcasted_iota(jnp.int32, (rb, rb), 1)
    ltri = (ri > ci).astype(jnp.float32)
    lane = lax.broadcasted_iota(jnp.int32, (rb, n_exp), 1)
    ohs, prs, sums = [], [], []
    for b in range(nblk):
        ohb = (lane == ep[b * rb:(b + 1) * rb, :]).astype(jnp.float32)
        ohs.append(ohb)
        prs.append(lax.dot_general(ltri, ohb, (((1,), (0,)), ((), ())),
                                   preferred_element_type=jnp.float32))
        sums.append(jnp.sum(ohb, axis=0, keepdims=True))
    bs = jnp.concatenate(sums, axis=0)
    bri = lax.broadcasted_iota(jnp.int32, (nblk, nblk), 0)
    bci = lax.broadcasted_iota(jnp.int32, (nblk, nblk), 1)
    btri = (bri > bci).astype(jnp.float32)
    boff = lax.dot_general(btri, bs, (((1,), (0,)), ((), ())),
                           preferred_element_type=jnp.float32)
    counts = jnp.sum(bs, axis=0, keepdims=True)
    nbe = (counts.astype(jnp.int32) + (tbg - 1)) // tbg
    eri = lax.broadcasted_iota(jnp.int32, (n_exp, n_exp), 0)
    eci = lax.broadcasted_iota(jnp.int32, (n_exp, n_exp), 1)
    ustri = (eri < eci).astype(jnp.float32)
    startb = lax.dot_general(nbe.astype(jnp.float32), ustri,
                             (((1,), (0,)), ((), ())),
                             preferred_element_type=jnp.float32)
    pbase = startb * tbg
    used_ref[...] = jnp.sum(nbe, axis=1, keepdims=True)
    for b in range(nblk):
        pos = prs[b] + boff[b:b + 1, :] + pbase
        destb = jnp.sum(ohs[b] * pos, axis=1, keepdims=True)
        dest_ref[b * rb:(b + 1) * rb, :] = destb.astype(jnp.int32)
    bi = lax.broadcasted_iota(jnp.int32, (nb, n_exp), 0).astype(jnp.float32)
    be_ref[...] = (jnp.sum((bi >= startb).astype(jnp.int32), axis=1,
                           keepdims=True) - 1)


def _meta(x, wg, tbg, nb, interpret=False):
    t, d = x.shape
    n_exp = wg.shape[0]
    return pl.pallas_call(
        functools.partial(_meta_body, n_exp, tbg, nb),
        out_shape=(
            jax.ShapeDtypeStruct((2 * t, 1), jnp.int32),
            jax.ShapeDtypeStruct((t, 1), jnp.float32),
            jax.ShapeDtypeStruct((t, 1), jnp.float32),
            jax.ShapeDtypeStruct((nb, 1), jnp.int32),
            jax.ShapeDtypeStruct((1, 1), jnp.int32),
        ),
        interpret=interpret,
    )(x, wg)


def _sc_dispatch(x3, dest, tok, pt):
    p2 = dest.shape[0]
    nw = _NC * _NS
    rows_pw = p2 // nw
    nch = rows_pw // _CH_D
    mesh = plsc.VectorSubcoreMesh(core_axis_name="c", subcore_axis_name="s")

    @functools.partial(
        pl.kernel, mesh=mesh,
        out_type=jax.ShapeDtypeStruct((pt,) + x3.shape[1:], x3.dtype),
        scratch_types=[
            pltpu.VMEM((rows_pw,), jnp.int32),
            pltpu.VMEM((_CH_D,), jnp.int32),
            pltpu.VMEM((_CH_D,), jnp.int32),
            pltpu.VMEM((_CH_D,) + x3.shape[1:], x3.dtype),
            pltpu.VMEM((_CH_D,) + x3.shape[1:], x3.dtype),
            pltpu.SemaphoreType.DMA,
            pltpu.SemaphoreType.DMA,
            pltpu.SemaphoreType.DMA,
            pltpu.SemaphoreType.DMA,
        ])
    def k(x_hbm, dest_hbm, tok_hbm, xs_hbm, tchunk, dc0, dc1, rb0, rb1,
          sg0, sg1, ss0, ss1):
        cid = lax.axis_index("c")
        sid = lax.axis_index("s")
        wid = sid * _NC + cid
        rowbase = wid * rows_pw
        pltpu.sync_copy(tok_hbm.at[pl.ds(rowbase, rows_pw)], tchunk)
        bufs = [(dc0, rb0, sg0, ss0), (dc1, rb1, sg1, ss1)]

        def issue_gather(kk):
            dc, rb, sg, _ = bufs[kk % 2]
            pltpu.sync_copy(dest_hbm.at[pl.ds(rowbase + kk * _CH_D, _CH_D)], dc)
            return pltpu.async_copy(
                x_hbm.at[tchunk.at[pl.ds(kk * _CH_D, _CH_D)]], rb, sg)

        gathers = {0: issue_gather(0)}
        if nch > 1:
            gathers[1] = issue_gather(1)
        scatters = {}
        for kk in range(nch):
            dc, rb, _, ss = bufs[kk % 2]
            gathers[kk].wait()
            scatters[kk] = pltpu.async_copy(rb, xs_hbm.at[dc], ss)
            if kk + 2 < nch:
                scatters[kk].wait()
                gathers[kk + 2] = issue_gather(kk + 2)
        for kk in range(max(0, nch - 2), nch):
            scatters[kk].wait()

    return k(x3, dest, tok)


def _sc_gather(ys3, dest):
    p2 = dest.shape[0]
    nw = _NC * _NS
    rows_pw = p2 // nw
    nch = rows_pw // _CH_G
    mesh = plsc.VectorSubcoreMesh(core_axis_name="c", subcore_axis_name="s")

    @functools.partial(
        pl.kernel, mesh=mesh,
        out_type=jax.ShapeDtypeStruct((p2,) + ys3.shape[1:], ys3.dtype),
        scratch_types=[
            pltpu.VMEM((rows_pw,), jnp.int32),
            pltpu.VMEM((_CH_G,) + ys3.shape[1:], ys3.dtype),
            pltpu.VMEM((_CH_G,) + ys3.shape[1:], ys3.dtype),
            pltpu.SemaphoreType.DMA,
            pltpu.SemaphoreType.DMA,
            pltpu.SemaphoreType.DMA,
            pltpu.SemaphoreType.DMA,
        ])
    def k(ys_hbm, dest_hbm, yg_hbm, idx_v, rb0, rb1, sg0, sg1, ss0, ss1):
        cid = lax.axis_index("c")
        sid = lax.axis_index("s")
        wid = sid * _NC + cid
        rowbase = wid * rows_pw
        pltpu.sync_copy(dest_hbm.at[pl.ds(rowbase, rows_pw)], idx_v)
        bufs = [(rb0, sg0, ss0), (rb1, sg1, ss1)]

        def issue_gather(kk):
            rb, sg, _ = bufs[kk % 2]
            return pltpu.async_copy(
                ys_hbm.at[idx_v.at[pl.ds(kk * _CH_G, _CH_G)]], rb, sg)

        gathers = {0: issue_gather(0)}
        if nch > 1:
            gathers[1] = issue_gather(1)
        writes = {}
        for kk in range(nch):
            rb, _, ss = bufs[kk % 2]
            gathers[kk].wait()
            writes[kk] = pltpu.async_copy(
                rb, yg_hbm.at[pl.ds(rowbase + kk * _CH_G, _CH_G)], ss)
            if kk + 2 < nch:
                writes[kk].wait()
                gathers[kk + 2] = issue_gather(kk + 2)
        for kk in range(max(0, nch - 2), nch):
            writes[kk].wait()

    return k(ys3, dest)


def _gemm_body(tbg, ncc, sp_ref, xs_ref, w1_ref, w3_ref, w2_ref, ys_ref,
               w1b, w3b, w2b):
    b = pl.program_id(0)
    d = ncc * 128

    @pl.when(jnp.logical_or(b == 0, sp_ref[b] != sp_ref[b + 1]))
    def _():
        w1b[...] = w1_ref[0].astype(jnp.bfloat16)
        w3b[...] = w3_ref[0].astype(jnp.bfloat16)
        w2b[...] = w2_ref[0].astype(jnp.bfloat16)

    @pl.when(b < sp_ref[0])
    def _():
        xt = jnp.reshape(xs_ref[...], (tbg, d)).astype(jnp.bfloat16)
        h1 = _dot_t(xt, w1b[...])
        h3 = _dot_t(xt, w3b[...])
        g = (h1 * lax.logistic(h1) * h3).astype(jnp.bfloat16)
        ys_ref[...] = jnp.reshape(_dot_t(g, w2b[...]), (tbg, ncc, 128))


def _gemm(sp, xs3, w1, w3, w2, tbg, interpret=False):
    pt = xs3.shape[0]
    n_exp, h, d = w1.shape
    ncc = d // 128
    nb = pt // tbg
    grid_spec = pltpu.PrefetchScalarGridSpec(
        num_scalar_prefetch=1,
        grid=(nb,),
        in_specs=[
            pl.BlockSpec((tbg, ncc, 128), lambda b, sp: (b, 0, 0)),
            pl.BlockSpec((1, h, d), lambda b, sp: (sp[b + 1], 0, 0)),
            pl.BlockSpec((1, h, d), lambda b, sp: (sp[b + 1], 0, 0)),
            pl.BlockSpec((1, d, h), lambda b, sp: (sp[b + 1], 0, 0)),
        ],
        out_specs=pl.BlockSpec((tbg, ncc, 128), lambda b, sp: (b, 0, 0)),
        scratch_shapes=[
            pltpu.VMEM((h, d), jnp.bfloat16),
            pltpu.VMEM((h, d), jnp.bfloat16),
            pltpu.VMEM((d, h), jnp.bfloat16),
        ],
    )
    return pl.pallas_call(
        functools.partial(_gemm_body, tbg, ncc),
        grid_spec=grid_spec,
        out_shape=jax.ShapeDtypeStruct((pt, ncc, 128), jnp.float32),
        interpret=interpret,
    )(sp, xs3, w1, w3, w2)


def _combine_body(tb, d, ya_ref, yb_ref, m1_ref, m2_ref, x_ref, ws1_ref,
                  ws3_ref, ws2_ref, z_ref, w1b, w3b, w2b):
    i = pl.program_id(0)

    @pl.when(i == 0)
    def _():
        w1b[...] = ws1_ref[...].astype(jnp.bfloat16)
        w3b[...] = ws3_ref[...].astype(jnp.bfloat16)
        w2b[...] = ws2_ref[...].astype(jnp.bfloat16)

    xt = x_ref[...].astype(jnp.bfloat16)
    h1 = _dot_t(xt, w1b[...])
    h3 = _dot_t(xt, w3b[...])
    g = (h1 * lax.logistic(h1) * h3).astype(jnp.bfloat16)
    sh = _dot_t(g, w2b[...])
    ya = jnp.reshape(ya_ref[...], (tb, d))
    yb = jnp.reshape(yb_ref[...], (tb, d))
    z_ref[...] = m1_ref[...] * ya + m2_ref[...] * yb + sh


def _combine(yg3, m1, m2, x, ws1, ws3, ws2, interpret=False):
    t, d = x.shape
    sh_h = ws1.shape[0]
    ncc = d // 128
    tb = min(1024, t)
    nts = t // tb
    return pl.pallas_call(
        functools.partial(_combine_body, tb, d),
        grid=(nts,),
        in_specs=[
            pl.BlockSpec((tb, ncc, 128), lambda i: (i, 0, 0)),
            pl.BlockSpec((tb, ncc, 128), lambda i, _n=nts: (i + _n, 0, 0)),
            pl.BlockSpec((tb, 1), lambda i: (i, 0)),
            pl.BlockSpec((tb, 1), lambda i: (i, 0)),
            pl.BlockSpec((tb, d), lambda i: (i, 0)),
            pl.BlockSpec((sh_h, d), lambda i: (0, 0)),
            pl.BlockSpec((sh_h, d), lambda i: (0, 0)),
            pl.BlockSpec((d, sh_h), lambda i: (0, 0)),
        ],
        out_specs=pl.BlockSpec((tb, d), lambda i: (i, 0)),
        out_shape=jax.ShapeDtypeStruct((t, d), jnp.float32),
        scratch_shapes=[
            pltpu.VMEM((sh_h, d), jnp.bfloat16),
            pltpu.VMEM((sh_h, d), jnp.bfloat16),
            pltpu.VMEM((d, sh_h), jnp.bfloat16),
        ],
        interpret=interpret,
    )(yg3, yg3, m1, m2, x, ws1, ws3, ws2)


def kernel(x, Wg, w1, w3, w2, ws1, ws3, ws2):
    t, d = x.shape
    n_exp = Wg.shape[0]
    k_act = 2
    p2 = k_act * t
    pt = p2 + n_exp * _TBG
    nb = pt // _TBG

    dest, m1, m2, be, used = _meta(x, Wg, _TBG, nb)
    sp = jnp.concatenate([used.reshape(-1), be.reshape(-1)])
    tok = jnp.concatenate([jnp.arange(t, dtype=jnp.int32)] * k_act)
    x3 = x.reshape(t, d // 128, 128)
    xs3 = _sc_dispatch(x3, dest.reshape(-1), tok, pt)
    ys3 = _gemm(sp, xs3, w1, w3, w2, _TBG)
    yg3 = _sc_gather(ys3, dest.reshape(-1))
    return _combine(yg3, m1, m2, x, ws1, ws3, ws2)
